# baseline (speedup 1.0000x reference)
import functools
import jax
import jax.numpy as jnp
from jax.experimental import pallas as pl
from jax.experimental.pallas import tpu as pltpu
from jax.experimental.pallas import tpu_sc as plsc

NCLS = 21
THR = 0.5
NEGPOS = 3
PP = 25500
RR = 200
LL = 128
PPAD = RR * LL
NT = 10
NOUT = 8
NCH = PPAD // 16
INT_MIN = -2147483648
INT_MAX = 2147483647


def _smooth_l1(d):
    ad = jnp.abs(d)
    return jnp.where(ad < 1.0, 0.5 * d * d, ad - 0.5)


def _loss_body(tgt_ref, pri_ref, loc_ref, conf_ref, obj_ref,
               out_ref, minec_ref, mineo_ref):
    b = pl.program_id(0)

    pcx = pri_ref[0]
    pcy = pri_ref[1]
    pw = pri_ref[2]
    ph = pri_ref[3]
    px1 = pcx - pw * 0.5
    py1 = pcy - ph * 0.5
    px2 = pcx + pw * 0.5
    py2 = pcy + ph * 0.5
    parea = (px2 - px1) * (py2 - py1)

    row = jax.lax.broadcasted_iota(jnp.int32, (RR, LL), 0)
    col = jax.lax.broadcasted_iota(jnp.int32, (RR, LL), 1)
    p_idx = row * LL + col
    valid = p_idx < PP

    bto = jnp.full((RR, LL), -1.0, jnp.float32)
    bti = jnp.zeros((RR, LL), jnp.int32)
    tx1 = [tgt_ref[b, 5 * j + 0] for j in range(NT)]
    ty1 = [tgt_ref[b, 5 * j + 1] for j in range(NT)]
    tx2 = [tgt_ref[b, 5 * j + 2] for j in range(NT)]
    ty2 = [tgt_ref[b, 5 * j + 3] for j in range(NT)]
    tlab = [tgt_ref[b, 5 * j + 4] for j in range(NT)]
    bpi = []
    for j in range(NT):
        ix = jnp.maximum(jnp.minimum(px2, tx2[j]) - jnp.maximum(px1, tx1[j]), 0.0)
        iy = jnp.maximum(jnp.minimum(py2, ty2[j]) - jnp.maximum(py1, ty1[j]), 0.0)
        inter = ix * iy
        tarea = (tx2[j] - tx1[j]) * (ty2[j] - ty1[j])
        ov = inter / (tarea + parea - inter)
        ov = jnp.where(valid, ov, -1.0)
        upd = ov > bto
        bto = jnp.where(upd, ov, bto)
        bti = jnp.where(upd, j, bti)
        mj = jnp.max(ov)
        bpi.append(jnp.min(jnp.where(ov == mj, p_idx, PPAD)))
    for j in range(NT):
        hit = p_idx == bpi[j]
        bto = jnp.where(hit, 2.0, bto)
        bti = jnp.where(hit, j, bti)
    mx1 = jnp.zeros((RR, LL), jnp.float32)
    my1 = jnp.zeros((RR, LL), jnp.float32)
    mx2 = jnp.zeros((RR, LL), jnp.float32)
    my2 = jnp.zeros((RR, LL), jnp.float32)
    lab = jnp.zeros((RR, LL), jnp.float32)
    for j in range(NT):
        sel = bti == j
        mx1 = jnp.where(sel, tx1[j], mx1)
        my1 = jnp.where(sel, ty1[j], my1)
        mx2 = jnp.where(sel, tx2[j], mx2)
        my2 = jnp.where(sel, ty2[j], my2)
        lab = jnp.where(sel, tlab[j], lab)
    conf_t = jnp.where(bto < THR, 0, (lab + 1.0).astype(jnp.int32))
    pos = conf_t > 0

    g0 = ((mx1 + mx2) * 0.5 - pcx) / (0.1 * pw)
    g1 = ((my1 + my2) * 0.5 - pcy) / (0.1 * ph)
    g2 = jnp.log((mx2 - mx1) / pw) / 0.2
    g3 = jnp.log((my2 - my1) / ph) / 0.2
    sl1 = (_smooth_l1(loc_ref[0, 0] - g0) + _smooth_l1(loc_ref[0, 1] - g1)
           + _smooth_l1(loc_ref[0, 2] - g2) + _smooth_l1(loc_ref[0, 3] - g3))
    loss_l = jnp.sum(jnp.where(pos, sl1, 0.0))

    o0 = obj_ref[0, 0]
    o1 = obj_ref[0, 1]
    mo = jnp.maximum(o0, o1)
    lse_o = jnp.log(jnp.exp(o0 - mo) + jnp.exp(o1 - mo)) + mo
    ce_o = lse_o - jnp.where(pos, o1, o0)

    mc = conf_ref[0, 0]
    for i in range(1, NCLS):
        mc = jnp.maximum(mc, conf_ref[0, i])
    s = jnp.zeros((RR, LL), jnp.float32)
    chosen = jnp.zeros((RR, LL), jnp.float32)
    for i in range(NCLS):
        ci = conf_ref[0, i]
        s = s + jnp.exp(ci - mc)
        chosen = jnp.where(conf_t == i, ci, chosen)
    ce_c = (jnp.log(s) + mc) - chosen

    minec_ref[0] = jnp.where(valid, jnp.where(pos, 0.0, ce_c), -1.0)
    mineo_ref[0] = jnp.where(valid, jnp.where(pos, 0.0, ce_o), -1.0)

    np_cnt = jnp.sum(pos.astype(jnp.int32))
    k = jnp.minimum(NEGPOS * np_cnt, PP - 1)

    out_ref[0, 0, 0] = loss_l
    out_ref[0, 0, 1] = jnp.sum(jnp.where(pos, ce_c, 0.0))
    out_ref[0, 0, 2] = jnp.sum(jnp.where(pos, ce_o, 0.0))
    out_ref[0, 0, 3] = np_cnt.astype(jnp.float32)
    out_ref[0, 0, 4] = k.astype(jnp.float32)
    out_ref[0, 0, 5] = 0.0
    out_ref[0, 0, 6] = 0.0
    out_ref[0, 0, 7] = 0.0


def _mine_body(vic_hbm, vio_hbm, k_hbm, out_hbm, vc_ref, vo_ref, kv_ref,
               res_ref, acc_c_ref, acc_o_ref, sem):
    wid = jax.lax.axis_index("s") * 2 + jax.lax.axis_index("c")
    pltpu.sync_copy(vic_hbm.at[wid], vc_ref)
    pltpu.sync_copy(vio_hbm.at[wid], vo_ref)
    pltpu.sync_copy(k_hbm.at[wid], kv_ref)
    lanes = jax.lax.iota(jnp.int32, 16)
    k = kv_ref[...]

    def _xl(x, op):
        for st in (8, 4, 2, 1):
            x = op(x, x.at[lanes ^ st].get(mode='promise_in_bounds'))
        return x

    def mx_body(i, st):
        mc, mo = st
        return (jnp.maximum(mc, vc_ref[pl.ds(i * 16, 16)]),
                jnp.maximum(mo, vo_ref[pl.ds(i * 16, 16)]))

    def _lane0(x):
        return jax.lax.squeeze(jax.lax.slice(x, (0,), (1,)), (0,))

    mcv, mov = jax.lax.fori_loop(
        0, NCH, mx_body,
        (jnp.full((16,), INT_MIN, jnp.int32), jnp.full((16,), INT_MIN, jnp.int32)))
    hi_c0 = _lane0(_xl(mcv, jnp.maximum))
    hi_o0 = _lane0(_xl(mov, jnp.maximum))

    def counts(mid_c, mid_o):
        acc_c_ref[...] = jnp.zeros((16,), jnp.float32)
        acc_o_ref[...] = jnp.zeros((16,), jnp.float32)

        def c_body(i, _):
            vc = vc_ref[pl.ds(i * 16, 16)]
            vo = vo_ref[pl.ds(i * 16, 16)]
            acc_c_ref[...] = acc_c_ref[...] + jnp.where(vc >= mid_c, 1.0, 0.0)
            acc_o_ref[...] = acc_o_ref[...] + jnp.where(vo >= mid_o, 1.0, 0.0)
            return 0

        jax.lax.fori_loop(0, NCH, c_body, 0)
        return (_lane0(_xl(acc_c_ref[...], jnp.add)),
                _lane0(_xl(acc_o_ref[...], jnp.add)))

    def _done(lo, hi, cnt):
        return jnp.logical_or(cnt == k_s, lo == hi)

    def bis(_, st):
        lo_c, hi_c, cnt_c, lo_o, hi_o, cnt_o = st
        mid_c = lo_c + jax.lax.shift_right_arithmetic(hi_c - lo_c + 1, 1)
        mid_o = lo_o + jax.lax.shift_right_arithmetic(hi_o - lo_o + 1, 1)
        cc, co = counts(mid_c, mid_o)
        dc = _done(lo_c, hi_c, cnt_c)
        do = _done(lo_o, hi_o, cnt_o)
        ok_c = cc >= k_s
        lo_c2 = jnp.where(dc, lo_c, jnp.where(ok_c, mid_c, lo_c))
        hi_c2 = jnp.where(dc, hi_c, jnp.where(ok_c, hi_c, mid_c - 1))
        cnt_c2 = jnp.where(dc, cnt_c, jnp.where(ok_c, cc, cnt_c))
        ok_o = co >= k_s
        lo_o2 = jnp.where(do, lo_o, jnp.where(ok_o, mid_o, lo_o))
        hi_o2 = jnp.where(do, hi_o, jnp.where(ok_o, hi_o, mid_o - 1))
        cnt_o2 = jnp.where(do, cnt_o, jnp.where(ok_o, co, cnt_o))
        return (lo_c2, hi_c2, cnt_c2, lo_o2, hi_o2, cnt_o2)

    k_s = _lane0(k)
    big = jnp.float32(PPAD + 1)
    lo_c, _, cnt_c, lo_o, _, cnt_o = jax.lax.fori_loop(
        0, 33, bis,
        (jnp.int32(0), hi_c0, big, jnp.int32(0), hi_o0, big))

    def mn_body(i, st):
        mc, mo = st
        vc = vc_ref[pl.ds(i * 16, 16)]
        vo = vo_ref[pl.ds(i * 16, 16)]
        return (jnp.minimum(mc, jnp.where(vc >= lo_c, vc, INT_MAX)),
                jnp.minimum(mo, jnp.where(vo >= lo_o, vo, INT_MAX)))

    mnc, mno = jax.lax.fori_loop(
        0, NCH, mn_body,
        (jnp.full((16,), INT_MAX, jnp.int32), jnp.full((16,), INT_MAX, jnp.int32)))
    t_c = jnp.where(cnt_c == k_s, _lane0(_xl(mnc, jnp.minimum)), lo_c)
    t_o = jnp.where(cnt_o == k_s, _lane0(_xl(mno, jnp.minimum)), lo_o)

    def fin_body(i, st):
        gc, sc, go, so = st
        vc = vc_ref[pl.ds(i * 16, 16)]
        vo = vo_ref[pl.ds(i * 16, 16)]
        gtc = vc > t_c
        gto = vo > t_o
        vcf = jax.lax.bitcast_convert_type(vc, jnp.float32)
        vof = jax.lax.bitcast_convert_type(vo, jnp.float32)
        return (gc + jnp.where(gtc, 1.0, 0.0), sc + jnp.where(gtc, vcf, 0.0),
                go + jnp.where(gto, 1.0, 0.0), so + jnp.where(gto, vof, 0.0))

    z_f = jnp.zeros((16,), jnp.float32)
    gcv, scv, gov, sov = jax.lax.fori_loop(0, NCH, fin_body,
                                           (z_f, z_f, z_f, z_f))

    t_c_f = jax.lax.bitcast_convert_type(jnp.full((16,), t_c, jnp.int32),
                                         jnp.float32)
    t_o_f = jax.lax.bitcast_convert_type(jnp.full((16,), t_o, jnp.int32),
                                         jnp.float32)
    res = jnp.where(lanes == 0, t_c_f, 0.0)
    res = res + jnp.where(lanes == 1, _xl(gcv, jnp.add), 0.0)
    res = res + jnp.where(lanes == 2, _xl(scv, jnp.add), 0.0)
    res = res + jnp.where(lanes == 3, t_o_f, 0.0)
    res = res + jnp.where(lanes == 4, _xl(gov, jnp.add), 0.0)
    res = res + jnp.where(lanes == 5, _xl(sov, jnp.add), 0.0)
    res_ref[...] = res
    pltpu.sync_copy(res_ref, out_hbm.at[wid])


def _mine_sc(vi_c, vi_o, kvec):
    fn = pl.kernel(
        _mine_body,
        mesh=plsc.VectorSubcoreMesh(core_axis_name="c", subcore_axis_name="s"),
        out_type=jax.ShapeDtypeStruct((32, 16), jnp.float32),
        scratch_types=[
            pltpu.VMEM((PPAD,), jnp.int32),
            pltpu.VMEM((PPAD,), jnp.int32),
            pltpu.VMEM((16,), jnp.float32),
            pltpu.VMEM((16,), jnp.float32),
            pltpu.VMEM((16,), jnp.float32),
            pltpu.VMEM((16,), jnp.float32),
            pltpu.SemaphoreType.DMA,
        ],
    )
    return fn(vi_c, vi_o, kvec)


def kernel(loc_data, conf_data, obj_data, priors, targets):
    bsz = loc_data.shape[0]
    pad = PPAD - PP

    def prep(x):
        x = jnp.pad(x, ((0, 0), (0, pad), (0, 0)))
        return x.transpose(0, 2, 1).reshape(bsz, x.shape[2], RR, LL)

    loc_p = prep(loc_data)
    conf_p = prep(conf_data)
    obj_p = prep(obj_data)
    pri_p = jnp.pad(priors, ((0, pad), (0, 0))).T.reshape(4, RR, LL)
    tgt = targets.reshape(bsz, NT * 5)

    out, mine_c, mine_o = pl.pallas_call(
        _loss_body,
        grid=(bsz,),
        in_specs=[
            pl.BlockSpec(memory_space=pltpu.SMEM),
            pl.BlockSpec((4, RR, LL), lambda b: (0, 0, 0)),
            pl.BlockSpec((1, 4, RR, LL), lambda b: (b, 0, 0, 0)),
            pl.BlockSpec((1, NCLS, RR, LL), lambda b: (b, 0, 0, 0)),
            pl.BlockSpec((1, 2, RR, LL), lambda b: (b, 0, 0, 0)),
        ],
        out_specs=[
            pl.BlockSpec((1, 1, NOUT), lambda b: (b, 0, 0),
                         memory_space=pltpu.SMEM),
            pl.BlockSpec((1, RR, LL), lambda b: (b, 0, 0)),
            pl.BlockSpec((1, RR, LL), lambda b: (b, 0, 0)),
        ],
        out_shape=[
            jax.ShapeDtypeStruct((bsz, 1, NOUT), jnp.float32),
            jax.ShapeDtypeStruct((bsz, RR, LL), jnp.float32),
            jax.ShapeDtypeStruct((bsz, RR, LL), jnp.float32),
        ],
    )(tgt, pri_p, loc_p, conf_p, obj_p)

    o = out.reshape(bsz, NOUT)
    vi_c = jax.lax.bitcast_convert_type(mine_c.reshape(bsz, PPAD), jnp.int32)
    vi_o = jax.lax.bitcast_convert_type(mine_o.reshape(bsz, PPAD), jnp.int32)
    ksplat = jnp.broadcast_to(o[:, 4:5], (bsz, 16))
    sc = _mine_sc(vi_c, vi_o, ksplat)

    kf = o[:, 4]
    neg_c = sc[:, 2] + (kf - sc[:, 1]) * sc[:, 0]
    neg_o = sc[:, 5] + (kf - sc[:, 4]) * sc[:, 3]

    n_pos = jnp.maximum(jnp.sum(o[:, 3]), 1.0)
    n_neg = jnp.maximum(jnp.sum(kf), 1.0)
    loss_l = jnp.sum(o[:, 0]) / n_pos
    loss_c = jnp.sum(o[:, 1] + neg_c) / n_pos
    loss_obj = 0.4 * jnp.sum(o[:, 2] + neg_o) / n_neg
    return (loss_l, loss_c, loss_obj)


# speedup vs baseline: 1.5075x; 1.5075x over previous
import functools
import jax
import jax.numpy as jnp
from jax.experimental import pallas as pl
from jax.experimental.pallas import tpu as pltpu
from jax.experimental.pallas import tpu_sc as plsc

NCLS = 21
THR = 0.5
NEGPOS = 3
PP = 25500
RR = 200
LL = 128
PPAD = RR * LL
NT = 10
NOUT = 8
NCH = PPAD // 16
INT_MIN = -2147483648
INT_MAX = 2147483647


def _smooth_l1(d):
    ad = jnp.abs(d)
    return jnp.where(ad < 1.0, 0.5 * d * d, ad - 0.5)


def _loss_body(tgt_ref, pri_ref, loc_ref, conf_ref, obj_ref,
               out_ref, minec_ref, mineo_ref):
    b = pl.program_id(0)

    pcx = pri_ref[0]
    pcy = pri_ref[1]
    pw = pri_ref[2]
    ph = pri_ref[3]
    px1 = pcx - pw * 0.5
    py1 = pcy - ph * 0.5
    px2 = pcx + pw * 0.5
    py2 = pcy + ph * 0.5
    parea = (px2 - px1) * (py2 - py1)

    row = jax.lax.broadcasted_iota(jnp.int32, (RR, LL), 0)
    col = jax.lax.broadcasted_iota(jnp.int32, (RR, LL), 1)
    p_idx = row * LL + col
    valid = p_idx < PP

    bto = jnp.full((RR, LL), -1.0, jnp.float32)
    bti = jnp.zeros((RR, LL), jnp.int32)
    tx1 = [tgt_ref[b, 5 * j + 0] for j in range(NT)]
    ty1 = [tgt_ref[b, 5 * j + 1] for j in range(NT)]
    tx2 = [tgt_ref[b, 5 * j + 2] for j in range(NT)]
    ty2 = [tgt_ref[b, 5 * j + 3] for j in range(NT)]
    tlab = [tgt_ref[b, 5 * j + 4] for j in range(NT)]
    bpi = []
    for j in range(NT):
        ix = jnp.maximum(jnp.minimum(px2, tx2[j]) - jnp.maximum(px1, tx1[j]), 0.0)
        iy = jnp.maximum(jnp.minimum(py2, ty2[j]) - jnp.maximum(py1, ty1[j]), 0.0)
        inter = ix * iy
        tarea = (tx2[j] - tx1[j]) * (ty2[j] - ty1[j])
        ov = inter / (tarea + parea - inter)
        ov = jnp.where(valid, ov, -1.0)
        upd = ov > bto
        bto = jnp.where(upd, ov, bto)
        bti = jnp.where(upd, j, bti)
        mj = jnp.max(ov)
        bpi.append(jnp.min(jnp.where(ov == mj, p_idx, PPAD)))
    for j in range(NT):
        hit = p_idx == bpi[j]
        bto = jnp.where(hit, 2.0, bto)
        bti = jnp.where(hit, j, bti)
    mx1 = jnp.zeros((RR, LL), jnp.float32)
    my1 = jnp.zeros((RR, LL), jnp.float32)
    mx2 = jnp.zeros((RR, LL), jnp.float32)
    my2 = jnp.zeros((RR, LL), jnp.float32)
    lab = jnp.zeros((RR, LL), jnp.float32)
    for j in range(NT):
        sel = bti == j
        mx1 = jnp.where(sel, tx1[j], mx1)
        my1 = jnp.where(sel, ty1[j], my1)
        mx2 = jnp.where(sel, tx2[j], mx2)
        my2 = jnp.where(sel, ty2[j], my2)
        lab = jnp.where(sel, tlab[j], lab)
    conf_t = jnp.where(bto < THR, 0, (lab + 1.0).astype(jnp.int32))
    pos = conf_t > 0

    g0 = ((mx1 + mx2) * 0.5 - pcx) / (0.1 * pw)
    g1 = ((my1 + my2) * 0.5 - pcy) / (0.1 * ph)
    g2 = jnp.log((mx2 - mx1) / pw) / 0.2
    g3 = jnp.log((my2 - my1) / ph) / 0.2
    sl1 = (_smooth_l1(loc_ref[0, 0] - g0) + _smooth_l1(loc_ref[0, 1] - g1)
           + _smooth_l1(loc_ref[0, 2] - g2) + _smooth_l1(loc_ref[0, 3] - g3))
    loss_l = jnp.sum(jnp.where(pos, sl1, 0.0))

    o0 = obj_ref[0, 0]
    o1 = obj_ref[0, 1]
    mo = jnp.maximum(o0, o1)
    lse_o = jnp.log(jnp.exp(o0 - mo) + jnp.exp(o1 - mo)) + mo
    ce_o = lse_o - jnp.where(pos, o1, o0)

    mc = conf_ref[0, 0]
    for i in range(1, NCLS):
        mc = jnp.maximum(mc, conf_ref[0, i])
    s = jnp.zeros((RR, LL), jnp.float32)
    chosen = jnp.zeros((RR, LL), jnp.float32)
    for i in range(NCLS):
        ci = conf_ref[0, i]
        s = s + jnp.exp(ci - mc)
        chosen = jnp.where(conf_t == i, ci, chosen)
    ce_c = (jnp.log(s) + mc) - chosen

    minec_ref[0] = jnp.where(valid, jnp.where(pos, 0.0, ce_c), -1.0)
    mineo_ref[0] = jnp.where(valid, jnp.where(pos, 0.0, ce_o), -1.0)

    np_cnt = jnp.sum(pos.astype(jnp.int32))
    k = jnp.minimum(NEGPOS * np_cnt, PP - 1)

    out_ref[0, 0, 0] = loss_l
    out_ref[0, 0, 1] = jnp.sum(jnp.where(pos, ce_c, 0.0))
    out_ref[0, 0, 2] = jnp.sum(jnp.where(pos, ce_o, 0.0))
    out_ref[0, 0, 3] = np_cnt.astype(jnp.float32)
    out_ref[0, 0, 4] = k.astype(jnp.float32)
    out_ref[0, 0, 5] = 0.0
    out_ref[0, 0, 6] = 0.0
    out_ref[0, 0, 7] = 0.0


def _mine_body(vic_hbm, vio_hbm, k_hbm, out_hbm, vc_ref, vo_ref, kv_ref,
               res_ref, acc_c_ref, acc_o_ref, sem):
    wid = jax.lax.axis_index("s") * 2 + jax.lax.axis_index("c")
    pltpu.sync_copy(vic_hbm.at[wid], vc_ref)
    pltpu.sync_copy(vio_hbm.at[wid], vo_ref)
    pltpu.sync_copy(k_hbm.at[wid], kv_ref)
    lanes = jax.lax.iota(jnp.int32, 16)
    k = kv_ref[...]

    def _xl(x, op):
        for st in (8, 4, 2, 1):
            x = op(x, x.at[lanes ^ st].get(mode='promise_in_bounds'))
        return x

    def mx_body(i, st):
        mc, mo = st
        return (jnp.maximum(mc, vc_ref[pl.ds(i * 16, 16)]),
                jnp.maximum(mo, vo_ref[pl.ds(i * 16, 16)]))

    def _lane0(x):
        return jax.lax.squeeze(jax.lax.slice(x, (0,), (1,)), (0,))

    mcv, mov = jax.lax.fori_loop(
        0, NCH, mx_body,
        (jnp.full((16,), INT_MIN, jnp.int32), jnp.full((16,), INT_MIN, jnp.int32)))
    hi_c0 = _lane0(_xl(mcv, jnp.maximum))
    hi_o0 = _lane0(_xl(mov, jnp.maximum))

    def counts(mid_c, mid_o):
        acc_c_ref[...] = jnp.zeros((16,), jnp.float32)
        acc_o_ref[...] = jnp.zeros((16,), jnp.float32)

        def c_body(i, _):
            lc = jnp.zeros((16,), jnp.float32)
            lo = jnp.zeros((16,), jnp.float32)
            for u in range(8):
                vc = vc_ref[pl.ds((i * 8 + u) * 16, 16)]
                vo = vo_ref[pl.ds((i * 8 + u) * 16, 16)]
                lc = lc + jnp.where(vc >= mid_c, 1.0, 0.0)
                lo = lo + jnp.where(vo >= mid_o, 1.0, 0.0)
            acc_c_ref[...] = acc_c_ref[...] + lc
            acc_o_ref[...] = acc_o_ref[...] + lo
            return 0

        jax.lax.fori_loop(0, NCH // 8, c_body, 0)
        return (_lane0(_xl(acc_c_ref[...], jnp.add)),
                _lane0(_xl(acc_o_ref[...], jnp.add)))

    def _done(lo, hi, cnt):
        return jnp.logical_or(cnt == k_s, lo == hi)

    def bis(_, st):
        lo_c, hi_c, cnt_c, lo_o, hi_o, cnt_o = st
        mid_c = lo_c + jax.lax.shift_right_arithmetic(hi_c - lo_c + 1, 1)
        mid_o = lo_o + jax.lax.shift_right_arithmetic(hi_o - lo_o + 1, 1)
        cc, co = counts(mid_c, mid_o)
        dc = _done(lo_c, hi_c, cnt_c)
        do = _done(lo_o, hi_o, cnt_o)
        ok_c = cc >= k_s
        lo_c2 = jnp.where(dc, lo_c, jnp.where(ok_c, mid_c, lo_c))
        hi_c2 = jnp.where(dc, hi_c, jnp.where(ok_c, hi_c, mid_c - 1))
        cnt_c2 = jnp.where(dc, cnt_c, jnp.where(ok_c, cc, cnt_c))
        ok_o = co >= k_s
        lo_o2 = jnp.where(do, lo_o, jnp.where(ok_o, mid_o, lo_o))
        hi_o2 = jnp.where(do, hi_o, jnp.where(ok_o, hi_o, mid_o - 1))
        cnt_o2 = jnp.where(do, cnt_o, jnp.where(ok_o, co, cnt_o))
        return (lo_c2, hi_c2, cnt_c2, lo_o2, hi_o2, cnt_o2)

    k_s = _lane0(k)
    big = jnp.float32(PPAD + 1)
    lo_c, _, cnt_c, lo_o, _, cnt_o = jax.lax.fori_loop(
        0, 33, bis,
        (jnp.int32(0), hi_c0, big, jnp.int32(0), hi_o0, big))

    def mn_body(i, st):
        mc, mo = st
        vc = vc_ref[pl.ds(i * 16, 16)]
        vo = vo_ref[pl.ds(i * 16, 16)]
        return (jnp.minimum(mc, jnp.where(vc >= lo_c, vc, INT_MAX)),
                jnp.minimum(mo, jnp.where(vo >= lo_o, vo, INT_MAX)))

    mnc, mno = jax.lax.fori_loop(
        0, NCH, mn_body,
        (jnp.full((16,), INT_MAX, jnp.int32), jnp.full((16,), INT_MAX, jnp.int32)))
    t_c = jnp.where(cnt_c == k_s, _lane0(_xl(mnc, jnp.minimum)), lo_c)
    t_o = jnp.where(cnt_o == k_s, _lane0(_xl(mno, jnp.minimum)), lo_o)

    def fin_body(i, st):
        gc, sc, go, so = st
        vc = vc_ref[pl.ds(i * 16, 16)]
        vo = vo_ref[pl.ds(i * 16, 16)]
        gtc = vc > t_c
        gto = vo > t_o
        vcf = jax.lax.bitcast_convert_type(vc, jnp.float32)
        vof = jax.lax.bitcast_convert_type(vo, jnp.float32)
        return (gc + jnp.where(gtc, 1.0, 0.0), sc + jnp.where(gtc, vcf, 0.0),
                go + jnp.where(gto, 1.0, 0.0), so + jnp.where(gto, vof, 0.0))

    z_f = jnp.zeros((16,), jnp.float32)
    gcv, scv, gov, sov = jax.lax.fori_loop(0, NCH, fin_body,
                                           (z_f, z_f, z_f, z_f))

    t_c_f = jax.lax.bitcast_convert_type(jnp.full((16,), t_c, jnp.int32),
                                         jnp.float32)
    t_o_f = jax.lax.bitcast_convert_type(jnp.full((16,), t_o, jnp.int32),
                                         jnp.float32)
    res = jnp.where(lanes == 0, t_c_f, 0.0)
    res = res + jnp.where(lanes == 1, _xl(gcv, jnp.add), 0.0)
    res = res + jnp.where(lanes == 2, _xl(scv, jnp.add), 0.0)
    res = res + jnp.where(lanes == 3, t_o_f, 0.0)
    res = res + jnp.where(lanes == 4, _xl(gov, jnp.add), 0.0)
    res = res + jnp.where(lanes == 5, _xl(sov, jnp.add), 0.0)
    res_ref[...] = res
    pltpu.sync_copy(res_ref, out_hbm.at[wid])


def _mine_sc(vi_c, vi_o, kvec):
    fn = pl.kernel(
        _mine_body,
        mesh=plsc.VectorSubcoreMesh(core_axis_name="c", subcore_axis_name="s"),
        out_type=jax.ShapeDtypeStruct((32, 16), jnp.float32),
        scratch_types=[
            pltpu.VMEM((PPAD,), jnp.int32),
            pltpu.VMEM((PPAD,), jnp.int32),
            pltpu.VMEM((16,), jnp.float32),
            pltpu.VMEM((16,), jnp.float32),
            pltpu.VMEM((16,), jnp.float32),
            pltpu.VMEM((16,), jnp.float32),
            pltpu.SemaphoreType.DMA,
        ],
    )
    return fn(vi_c, vi_o, kvec)


def kernel(loc_data, conf_data, obj_data, priors, targets):
    bsz = loc_data.shape[0]
    pad = PPAD - PP

    def prep(x):
        x = jnp.pad(x, ((0, 0), (0, pad), (0, 0)))
        return x.transpose(0, 2, 1).reshape(bsz, x.shape[2], RR, LL)

    loc_p = prep(loc_data)
    conf_p = prep(conf_data)
    obj_p = prep(obj_data)
    pri_p = jnp.pad(priors, ((0, pad), (0, 0))).T.reshape(4, RR, LL)
    tgt = targets.reshape(bsz, NT * 5)

    out, mine_c, mine_o = pl.pallas_call(
        _loss_body,
        grid=(bsz,),
        in_specs=[
            pl.BlockSpec(memory_space=pltpu.SMEM),
            pl.BlockSpec((4, RR, LL), lambda b: (0, 0, 0)),
            pl.BlockSpec((1, 4, RR, LL), lambda b: (b, 0, 0, 0)),
            pl.BlockSpec((1, NCLS, RR, LL), lambda b: (b, 0, 0, 0)),
            pl.BlockSpec((1, 2, RR, LL), lambda b: (b, 0, 0, 0)),
        ],
        out_specs=[
            pl.BlockSpec((1, 1, NOUT), lambda b: (b, 0, 0),
                         memory_space=pltpu.SMEM),
            pl.BlockSpec((1, RR, LL), lambda b: (b, 0, 0)),
            pl.BlockSpec((1, RR, LL), lambda b: (b, 0, 0)),
        ],
        out_shape=[
            jax.ShapeDtypeStruct((bsz, 1, NOUT), jnp.float32),
            jax.ShapeDtypeStruct((bsz, RR, LL), jnp.float32),
            jax.ShapeDtypeStruct((bsz, RR, LL), jnp.float32),
        ],
    )(tgt, pri_p, loc_p, conf_p, obj_p)

    o = out.reshape(bsz, NOUT)
    vi_c = jax.lax.bitcast_convert_type(mine_c.reshape(bsz, PPAD), jnp.int32)
    vi_o = jax.lax.bitcast_convert_type(mine_o.reshape(bsz, PPAD), jnp.int32)
    ksplat = jnp.broadcast_to(o[:, 4:5], (bsz, 16))
    sc = _mine_sc(vi_c, vi_o, ksplat)

    kf = o[:, 4]
    neg_c = sc[:, 2] + (kf - sc[:, 1]) * sc[:, 0]
    neg_o = sc[:, 5] + (kf - sc[:, 4]) * sc[:, 3]

    n_pos = jnp.maximum(jnp.sum(o[:, 3]), 1.0)
    n_neg = jnp.maximum(jnp.sum(kf), 1.0)
    loss_l = jnp.sum(o[:, 0]) / n_pos
    loss_c = jnp.sum(o[:, 1] + neg_c) / n_pos
    loss_obj = 0.4 * jnp.sum(o[:, 2] + neg_o) / n_neg
    return (loss_l, loss_c, loss_obj)


# speedup vs baseline: 1.6850x; 1.1178x over previous
import jax
import jax.numpy as jnp
from jax.experimental import pallas as pl
from jax.experimental.pallas import tpu as pltpu

NCLS = 21
THR = 0.5
NEGPOS = 3
PP = 25500
RR = 200
LL = 128
PPAD = RR * LL
NT = 10
NOUT = 8


def _smooth_l1(d):
    ad = jnp.abs(d)
    return jnp.where(ad < 1.0, 0.5 * d * d, ad - 0.5)


def _loss_body(tgt_ref, pri_ref, loc_ref, conf_ref, obj_ref, out_ref):
    b = pl.program_id(0)

    pcx = pri_ref[0]
    pcy = pri_ref[1]
    pw = pri_ref[2]
    ph = pri_ref[3]
    px1 = pcx - pw * 0.5
    py1 = pcy - ph * 0.5
    px2 = pcx + pw * 0.5
    py2 = pcy + ph * 0.5
    parea = (px2 - px1) * (py2 - py1)

    row = jax.lax.broadcasted_iota(jnp.int32, (RR, LL), 0)
    col = jax.lax.broadcasted_iota(jnp.int32, (RR, LL), 1)
    p_idx = row * LL + col
    valid = p_idx < PP

    bto = jnp.full((RR, LL), -1.0, jnp.float32)
    bti = jnp.zeros((RR, LL), jnp.int32)
    tx1 = [tgt_ref[b, 5 * j + 0] for j in range(NT)]
    ty1 = [tgt_ref[b, 5 * j + 1] for j in range(NT)]
    tx2 = [tgt_ref[b, 5 * j + 2] for j in range(NT)]
    ty2 = [tgt_ref[b, 5 * j + 3] for j in range(NT)]
    tlab = [tgt_ref[b, 5 * j + 4] for j in range(NT)]
    bpi = []
    for j in range(NT):
        ix = jnp.maximum(jnp.minimum(px2, tx2[j]) - jnp.maximum(px1, tx1[j]), 0.0)
        iy = jnp.maximum(jnp.minimum(py2, ty2[j]) - jnp.maximum(py1, ty1[j]), 0.0)
        inter = ix * iy
        tarea = (tx2[j] - tx1[j]) * (ty2[j] - ty1[j])
        ov = inter / (tarea + parea - inter)
        ov = jnp.where(valid, ov, -1.0)
        upd = ov > bto
        bto = jnp.where(upd, ov, bto)
        bti = jnp.where(upd, j, bti)
        mj = jnp.max(ov)
        bpi.append(jnp.min(jnp.where(ov == mj, p_idx, PPAD)))
    for j in range(NT):
        hit = p_idx == bpi[j]
        bto = jnp.where(hit, 2.0, bto)
        bti = jnp.where(hit, j, bti)
    mx1 = jnp.zeros((RR, LL), jnp.float32)
    my1 = jnp.zeros((RR, LL), jnp.float32)
    mx2 = jnp.zeros((RR, LL), jnp.float32)
    my2 = jnp.zeros((RR, LL), jnp.float32)
    lab = jnp.zeros((RR, LL), jnp.float32)
    for j in range(NT):
        sel = bti == j
        mx1 = jnp.where(sel, tx1[j], mx1)
        my1 = jnp.where(sel, ty1[j], my1)
        mx2 = jnp.where(sel, tx2[j], mx2)
        my2 = jnp.where(sel, ty2[j], my2)
        lab = jnp.where(sel, tlab[j], lab)
    conf_t = jnp.where(bto < THR, 0, (lab + 1.0).astype(jnp.int32))
    pos = conf_t > 0

    g0 = ((mx1 + mx2) * 0.5 - pcx) / (0.1 * pw)
    g1 = ((my1 + my2) * 0.5 - pcy) / (0.1 * ph)
    g2 = jnp.log((mx2 - mx1) / pw) / 0.2
    g3 = jnp.log((my2 - my1) / ph) / 0.2
    sl1 = (_smooth_l1(loc_ref[0, 0] - g0) + _smooth_l1(loc_ref[0, 1] - g1)
           + _smooth_l1(loc_ref[0, 2] - g2) + _smooth_l1(loc_ref[0, 3] - g3))
    loss_l = jnp.sum(jnp.where(pos, sl1, 0.0))

    o0 = obj_ref[0, 0]
    o1 = obj_ref[0, 1]
    mo = jnp.maximum(o0, o1)
    lse_o = jnp.log(jnp.exp(o0 - mo) + jnp.exp(o1 - mo)) + mo
    ce_o = lse_o - jnp.where(pos, o1, o0)

    mc = conf_ref[0, 0]
    for i in range(1, NCLS):
        mc = jnp.maximum(mc, conf_ref[0, i])
    s = jnp.zeros((RR, LL), jnp.float32)
    chosen = jnp.zeros((RR, LL), jnp.float32)
    for i in range(NCLS):
        ci = conf_ref[0, i]
        s = s + jnp.exp(ci - mc)
        chosen = jnp.where(conf_t == i, ci, chosen)
    ce_c = (jnp.log(s) + mc) - chosen

    mine_c = jnp.where(valid, jnp.where(pos, 0.0, ce_c), -1.0)
    mine_o = jnp.where(valid, jnp.where(pos, 0.0, ce_o), -1.0)
    vi_c = jax.lax.bitcast_convert_type(mine_c, jnp.int32)
    vi_o = jax.lax.bitcast_convert_type(mine_o, jnp.int32)
    np_cnt = jnp.sum(pos.astype(jnp.int32))
    k = jnp.minimum(NEGPOS * np_cnt, PP - 1)

    def eight_way(lo, hi, cnt, vi):
        s = (hi - lo + 8) // 8
        idx = jnp.int32(0)
        new_cnt = cnt
        for i in range(1, 8):
            ci = jnp.sum((vi >= lo + i * s).astype(jnp.int32))
            ok = ci >= k
            idx = idx + ok.astype(jnp.int32)
            new_cnt = jnp.where(ok, ci, new_cnt)
        new_lo = lo + idx * s
        new_hi = jnp.minimum(hi, lo + (idx + 1) * s - 1)
        return new_lo, new_hi, new_cnt

    def _done(lo, hi, cnt):
        return jnp.logical_or(cnt == k, lo == hi)

    def bis_cond(st):
        lo_c, hi_c, cnt_c, lo_o, hi_o, cnt_o = st
        return jnp.logical_not(jnp.logical_and(_done(lo_c, hi_c, cnt_c),
                                               _done(lo_o, hi_o, cnt_o)))

    def bis(st):
        lo_c, hi_c, cnt_c, lo_o, hi_o, cnt_o = st
        nl, nh, nc = eight_way(lo_c, hi_c, cnt_c, vi_c)
        d = _done(lo_c, hi_c, cnt_c)
        lo_c = jnp.where(d, lo_c, nl)
        hi_c = jnp.where(d, hi_c, nh)
        cnt_c = jnp.where(d, cnt_c, nc)
        nl, nh, nc = eight_way(lo_o, hi_o, cnt_o, vi_o)
        d = _done(lo_o, hi_o, cnt_o)
        lo_o = jnp.where(d, lo_o, nl)
        hi_o = jnp.where(d, hi_o, nh)
        cnt_o = jnp.where(d, cnt_o, nc)
        return (lo_c, hi_c, cnt_c, lo_o, hi_o, cnt_o)

    big = jnp.int32(PPAD + 1)
    lo_c, _, cnt_c, lo_o, _, cnt_o = jax.lax.while_loop(
        bis_cond, bis,
        (jnp.int32(0), jnp.max(vi_c), big, jnp.int32(0), jnp.max(vi_o), big))

    def pick_t(lo, cnt, vi):
        mn = jnp.min(jnp.where(vi >= lo, vi, jnp.int32(2147483647)))
        return jnp.where(cnt == k, mn, lo)

    t_c = pick_t(lo_c, cnt_c, vi_c)
    t_o = pick_t(lo_o, cnt_o, vi_o)

    def neg_sum(mine, t_int):
        t = jax.lax.bitcast_convert_type(t_int, jnp.float32)
        gt = mine > t
        g_cnt = jnp.sum(gt.astype(jnp.int32))
        return (jnp.sum(jnp.where(gt, mine, 0.0))
                + (k - g_cnt).astype(jnp.float32) * t)

    loss_c = jnp.sum(jnp.where(pos, ce_c, 0.0)) + neg_sum(mine_c, t_c)
    loss_o = jnp.sum(jnp.where(pos, ce_o, 0.0)) + neg_sum(mine_o, t_o)

    out_ref[0, 0, 0] = loss_l
    out_ref[0, 0, 1] = loss_c
    out_ref[0, 0, 2] = loss_o
    out_ref[0, 0, 3] = np_cnt.astype(jnp.float32)
    out_ref[0, 0, 4] = k.astype(jnp.float32)
    out_ref[0, 0, 5] = 0.0
    out_ref[0, 0, 6] = 0.0
    out_ref[0, 0, 7] = 0.0


def kernel(loc_data, conf_data, obj_data, priors, targets):
    bsz = loc_data.shape[0]
    pad = PPAD - PP

    def prep(x):
        x = jnp.pad(x, ((0, 0), (0, pad), (0, 0)))
        return x.transpose(0, 2, 1).reshape(bsz, x.shape[2], RR, LL)

    loc_p = prep(loc_data)
    conf_p = prep(conf_data)
    obj_p = prep(obj_data)
    pri_p = jnp.pad(priors, ((0, pad), (0, 0))).T.reshape(4, RR, LL)
    tgt = targets.reshape(bsz, NT * 5)

    out = pl.pallas_call(
        _loss_body,
        grid=(bsz,),
        in_specs=[
            pl.BlockSpec(memory_space=pltpu.SMEM),
            pl.BlockSpec((4, RR, LL), lambda b: (0, 0, 0)),
            pl.BlockSpec((1, 4, RR, LL), lambda b: (b, 0, 0, 0)),
            pl.BlockSpec((1, NCLS, RR, LL), lambda b: (b, 0, 0, 0)),
            pl.BlockSpec((1, 2, RR, LL), lambda b: (b, 0, 0, 0)),
        ],
        out_specs=pl.BlockSpec((1, 1, NOUT), lambda b: (b, 0, 0),
                               memory_space=pltpu.SMEM),
        out_shape=jax.ShapeDtypeStruct((bsz, 1, NOUT), jnp.float32),
    )(tgt, pri_p, loc_p, conf_p, obj_p)

    o = out.reshape(bsz, NOUT)
    n_pos = jnp.maximum(jnp.sum(o[:, 3]), 1.0)
    n_neg = jnp.maximum(jnp.sum(o[:, 4]), 1.0)
    loss_l = jnp.sum(o[:, 0]) / n_pos
    loss_c = jnp.sum(o[:, 1]) / n_pos
    loss_obj = 0.4 * jnp.sum(o[:, 2]) / n_neg
    return (loss_l, loss_c, loss_obj)
